# baseline (speedup 1.0000x reference)
import functools
import jax
import jax.numpy as jnp
from jax import lax
from jax.experimental import pallas as pl
from jax.experimental.pallas import tpu as pltpu
from jax.experimental.pallas import tpu_sc as plsc

N = 10000
C = 128
E = 320000
P = 320000

NC = 2
NS = 16
NW = NC * NS
CH = 128
NCHUNK = (E + NW * CH - 1) // (NW * CH)
E_PAD = NW * CH * NCHUNK
KA0, KA1 = 79, 79
KP0, KP1 = 79, 79
N_PAD = 10240
ROWS_PER_TILE = N_PAD // NS
B = 1024
GRID = N_PAD // B

def _wid():
    return lax.axis_index("s") * NC + lax.axis_index("c")


def _sc_deg_body(didx_hbm, out_hbm, acc, didx_v, ones_v):
    cid = lax.axis_index("c")
    sid = lax.axis_index("s")
    wid = _wid()

    def zfill(r, _):
        ones_v[r, :] = jnp.zeros((16,), jnp.float32)
        return 0
    lax.fori_loop(0, CH, zfill, 0)
    for t in range(ROWS_PER_TILE // CH):
        pltpu.sync_copy(ones_v, acc.at[pl.ds((sid * (ROWS_PER_TILE // CH) + t) * CH, CH)])

    def fill(r, _):
        ones_v[r, :] = jnp.zeros((16,), jnp.float32) + 1.0
        return 0
    lax.fori_loop(0, CH, fill, 0)
    plsc.subcore_barrier()

    def body(j, _):
        pltpu.sync_copy(didx_hbm.at[wid, j], didx_v)
        pltpu.sync_copy(ones_v, acc.at[didx_v], add=True)
        return 0
    lax.fori_loop(0, NCHUNK, body, 0)
    plsc.subcore_barrier()
    pltpu.sync_copy(acc.at[pl.ds(sid * ROWS_PER_TILE, ROWS_PER_TILE)],
                    out_hbm.at[cid, pl.ds(sid * ROWS_PER_TILE, ROWS_PER_TILE)])


def _sc_agg_body(u_hbm, sidx_hbm, didx_hbm, out_hbm, acc, sidx0, didx0, sidx1,
                 didx1, rows0, rows1, sem0, sem1):
    cid = lax.axis_index("c")
    sid = lax.axis_index("s")
    wid = _wid()
    nch = jnp.where(cid == 0, KA0, KA1)

    def zfill(r, _):
        for k in range(C // 16):
            rows0[r, pl.ds(k * 16, 16)] = jnp.zeros((16,), jnp.float32)
        return 0
    lax.fori_loop(0, CH, zfill, 0)
    for t in range(ROWS_PER_TILE // CH):
        pltpu.sync_copy(rows0, acc.at[pl.ds((sid * (ROWS_PER_TILE // CH) + t) * CH, CH)])
    plsc.subcore_barrier()

    def load_idx(j, si, di):
        pltpu.sync_copy(sidx_hbm.at[wid, j], si)
        pltpu.sync_copy(didx_hbm.at[wid, j], di)

    def fire(si, rows, sem):
        pltpu.async_copy(u_hbm.at[si], rows, sem)

    def wait(si, rows, sem):
        pltpu.make_async_copy(u_hbm.at[si], rows, sem).wait()

    load_idx(0, sidx0, didx0)
    fire(sidx0, rows0, sem0)

    def body(t, _):
        j0 = 2 * t
        load_idx(j0 + 1, sidx1, didx1)
        fire(sidx1, rows1, sem1)
        wait(sidx0, rows0, sem0)
        pltpu.sync_copy(rows0, acc.at[didx0], add=True)
        load_idx(j0 + 2, sidx0, didx0)
        fire(sidx0, rows0, sem0)
        wait(sidx1, rows1, sem1)
        pltpu.sync_copy(rows1, acc.at[didx1], add=True)
        return 0
    lax.fori_loop(0, (nch - 1) // 2, body, 0)
    wait(sidx0, rows0, sem0)
    pltpu.sync_copy(rows0, acc.at[didx0], add=True)

    plsc.subcore_barrier()
    pltpu.sync_copy(acc.at[pl.ds(sid * ROWS_PER_TILE, ROWS_PER_TILE)],
                    out_hbm.at[cid, pl.ds(sid * ROWS_PER_TILE, ROWS_PER_TILE)])


def _sc_pair_body(g_hbm, sidx_hbm, didx_hbm, bl1_hbm, wl2_hbm, out_hbm,
                  sidx_all, didx_all, s0, d0, s1, d1, out_all, bl1_v, wl2_v,
                  sem0, sem1):
    wid = _wid()
    nch = jnp.where(lax.axis_index("c") == 0, KP0, KP1)
    pltpu.sync_copy(bl1_hbm, bl1_v)
    pltpu.sync_copy(wl2_hbm, wl2_v)
    pltpu.sync_copy(sidx_hbm.at[wid], sidx_all)
    pltpu.sync_copy(didx_hbm.at[wid], didx_all)
    lanes = lax.iota(jnp.int32, 16)

    def fire(j, bs, bd, sem):
        pltpu.async_copy(g_hbm.at[sidx_all.at[j]], bs, sem)
        pltpu.async_copy(g_hbm.at[didx_all.at[j]], bd, sem)

    def wait(j, bs, bd, sem):
        pltpu.make_async_copy(g_hbm.at[sidx_all.at[j]], bs, sem).wait()
        pltpu.make_async_copy(g_hbm.at[didx_all.at[j]], bd, sem).wait()

    def compute(j, bs, bd):
        def pair16(p, _):
            tot = jnp.zeros((16,), jnp.float32)
            for ii in range(16):
                i = p * 16 + ii
                acc = jnp.zeros((16,), jnp.float32)
                for k in range(C // 16):
                    d = bd[i, pl.ds(k * 16, 16)]
                    s = bs[i, pl.ds(k * 16, 16)]
                    t = jnp.maximum(d - s + bl1_v[pl.ds(k * 16, 16)], 0.0)
                    acc = acc + t * wl2_v[pl.ds(k * 16, 16)]
                tot = jnp.where(lanes == ii, jnp.sum(acc), tot)
            out_all[pl.ds(j * CH + p * 16, 16)] = tot
            return 0
        lax.fori_loop(0, CH // 16, pair16, 0)

    fire(0, s0, d0, sem0)

    def body(t, _):
        j0 = 2 * t
        fire(j0 + 1, s1, d1, sem1)
        wait(j0, s0, d0, sem0)
        compute(j0, s0, d0)
        fire(j0 + 2, s0, d0, sem0)
        wait(j0 + 1, s1, d1, sem1)
        compute(j0 + 1, s1, d1)
        return 0
    lax.fori_loop(0, (nch - 1) // 2, body, 0)
    wait(nch - 1, s0, d0, sem0)
    compute(nch - 1, s0, d0)
    pltpu.sync_copy(out_all, out_hbm.at[wid])


@functools.lru_cache(maxsize=1)
def _get_sc_kernels():
    mesh = plsc.VectorSubcoreMesh(core_axis_name="c", subcore_axis_name="s",
                                  num_cores=NC, num_subcores=NS)
    cp = pltpu.CompilerParams(needs_layout_passes=False)
    deg = pl.kernel(
        _sc_deg_body,
        out_type=jax.ShapeDtypeStruct((NC, N_PAD, 16), jnp.float32),
        mesh=mesh, compiler_params=cp,
        scratch_types=[
            pltpu.VMEM_SHARED((N_PAD, 16), jnp.float32),
            pltpu.VMEM((CH,), jnp.int32),
            pltpu.VMEM((CH, 16), jnp.float32),
        ])
    agg = pl.kernel(
        _sc_agg_body,
        out_type=jax.ShapeDtypeStruct((NC, N_PAD, C), jnp.float32),
        mesh=mesh, compiler_params=cp,
        scratch_types=[
            pltpu.VMEM_SHARED((N_PAD, C), jnp.float32),
            pltpu.VMEM((CH,), jnp.int32),
            pltpu.VMEM((CH,), jnp.int32),
            pltpu.VMEM((CH,), jnp.int32),
            pltpu.VMEM((CH,), jnp.int32),
            pltpu.VMEM((CH, C), jnp.float32),
            pltpu.VMEM((CH, C), jnp.float32),
            pltpu.SemaphoreType.DMA,
            pltpu.SemaphoreType.DMA,
        ])
    pair = pl.kernel(
        _sc_pair_body,
        out_type=jax.ShapeDtypeStruct((NW, KP0 * CH), jnp.float32),
        mesh=mesh, compiler_params=cp,
        scratch_types=[
            pltpu.VMEM((KP0, CH), jnp.int32),
            pltpu.VMEM((KP0, CH), jnp.int32),
            pltpu.VMEM((CH, C), jnp.float32),
            pltpu.VMEM((CH, C), jnp.float32),
            pltpu.VMEM((CH, C), jnp.float32),
            pltpu.VMEM((CH, C), jnp.float32),
            pltpu.VMEM((KP0 * CH,), jnp.float32),
            pltpu.VMEM((C,), jnp.float32),
            pltpu.VMEM((C,), jnp.float32),
            pltpu.SemaphoreType.DMA,
            pltpu.SemaphoreType.DMA,
        ])
    return deg, agg, pair


def _tc_embed_body(x_ref, degp_ref, We1_ref, be1_ref, We2_ref, be2_ref,
                   Wr_ref, br_ref, Wc1_ref, x0_ref, u1_ref):
    xb = x_ref[...]
    h = jnp.dot(xb.reshape(B * 5, C), We1_ref[...],
                preferred_element_type=jnp.float32) + be1_ref[...]
    h = jnp.maximum(h, 0.0)
    h = jnp.dot(h, We2_ref[...], preferred_element_type=jnp.float32) + be2_ref[...]
    h = h.reshape(B, 5, C).sum(axis=1)
    h = jnp.dot(h, Wr_ref[...], preferred_element_type=jnp.float32) + br_ref[...]
    x0 = jnp.maximum(h, 0.0)
    deg = degp_ref[0, :, 0:1] + degp_ref[1, :, 0:1] + 1.0
    dinv = lax.rsqrt(deg)
    x0_ref[...] = x0
    u1_ref[...] = jnp.dot(x0, Wc1_ref[...], preferred_element_type=jnp.float32) * dinv


def _tc_mid_body(A_ref, u_ref, degp_ref, bc1_ref, Wc2_ref, u2_ref):
    deg = degp_ref[0, :, 0:1] + degp_ref[1, :, 0:1] + 1.0
    dinv = lax.rsqrt(deg)
    y = jnp.maximum(dinv * (A_ref[0] + A_ref[1] + u_ref[...]) + bc1_ref[...], 0.0)
    u2_ref[...] = jnp.dot(y, Wc2_ref[...], preferred_element_type=jnp.float32) * dinv


def _tc_fin_body(A_ref, u_ref, degp_ref, x0_ref, bc2_ref, Wl1_ref, g_ref):
    deg = degp_ref[0, :, 0:1] + degp_ref[1, :, 0:1] + 1.0
    dinv = lax.rsqrt(deg)
    y2 = dinv * (A_ref[0] + A_ref[1] + u_ref[...]) + bc2_ref[...]
    x0 = x0_ref[...]
    emb = jnp.maximum(x0 + y2, 0.0) + x0
    g_ref[...] = jnp.dot(emb, Wl1_ref[...], preferred_element_type=jnp.float32)


def _row_spec(shape3=False):
    if shape3:
        return pl.BlockSpec((B, 5, C), lambda i: (i, 0, 0))
    return pl.BlockSpec((B, C), lambda i: (i, 0))


_full = lambda s: pl.BlockSpec(s, lambda i: tuple(0 for _ in s))
_degp_spec = pl.BlockSpec((NC, B, 16), lambda i: (0, i, 0))
_A_spec = pl.BlockSpec((NC, B, C), lambda i: (0, i, 0))


def _split(idx, k0, k1, fill, off1=0):
    n0 = NS * k0 * CH
    n1 = NS * k1 * CH
    arr = jnp.concatenate([idx, jnp.full((n0 + n1 - idx.shape[0],), fill, jnp.int32)])
    p0 = arr[:n0].reshape(NS, k0, CH)
    p1 = arr[n0:].reshape(NS, k1, CH)
    p1 = jnp.pad(p1, ((0, 0), (0, k0 - k1), (0, 0)), constant_values=fill) + off1
    return jnp.stack([p0, p1], axis=1).reshape(NW, k0, CH)


def kernel(x, edge_index, src_idx, dst_idx, We1, be1, We2, be2, Wr, br,
           Wc1, bc1, Wc2, bc2, Wl1, bl1, Wl2, bl2):
    f32 = jnp.float32
    src = edge_index[0]
    dst = edge_index[1]
    epad = E_PAD - E
    dstp_deg = jnp.concatenate([dst, jnp.full((epad,), N, jnp.int32)]).reshape(NW, NCHUNK, CH)
    srcp = _split(src, KA0, KA1, N, off1=N_PAD)
    dstp = _split(dst, KA0, KA1, N)
    sip = _split(src_idx, KP0, KP1, 0, off1=N_PAD)
    dip = _split(dst_idx, KP0, KP1, 0, off1=N_PAD)
    xp = jnp.pad(x, ((0, N_PAD - N), (0, 0), (0, 0)))

    sc_deg, sc_agg, sc_pair = _get_sc_kernels()
    degp = sc_deg(dstp_deg)

    x0, u1 = pl.pallas_call(
        _tc_embed_body,
        grid=(GRID,),
        in_specs=[_row_spec(True), _degp_spec, _full((C, C)), _full((1, C)),
                  _full((C, C)), _full((1, C)), _full((C, C)), _full((1, C)),
                  _full((C, C))],
        out_specs=[_row_spec(), _row_spec()],
        out_shape=[jax.ShapeDtypeStruct((N_PAD, C), f32),
                   jax.ShapeDtypeStruct((N_PAD, C), f32)],
    )(xp, degp, We1, be1.reshape(1, C), We2, be2.reshape(1, C),
      Wr, br.reshape(1, C), Wc1)

    A1 = sc_agg(jnp.concatenate([u1, u1], axis=0), srcp, dstp)

    u2 = pl.pallas_call(
        _tc_mid_body,
        grid=(GRID,),
        in_specs=[_A_spec, _row_spec(), _degp_spec, _full((1, C)), _full((C, C))],
        out_specs=_row_spec(),
        out_shape=jax.ShapeDtypeStruct((N_PAD, C), f32),
    )(A1, u1, degp, bc1.reshape(1, C), Wc2)

    A2 = sc_agg(jnp.concatenate([u2, u2], axis=0), srcp, dstp)

    g = pl.pallas_call(
        _tc_fin_body,
        grid=(GRID,),
        in_specs=[_A_spec, _row_spec(), _degp_spec, _row_spec(), _full((1, C)),
                  _full((C, C))],
        out_specs=_row_spec(),
        out_shape=jax.ShapeDtypeStruct((N_PAD, C), f32),
    )(A2, u2, degp, x0, bc2.reshape(1, C), Wl1)

    o = sc_pair(jnp.concatenate([g, g], axis=0), sip, dip, bl1, Wl2.reshape(C))
    r = o.reshape(NS, NC, KP0 * CH)
    o_flat = jnp.concatenate([r[:, 0, :].reshape(-1),
                              r[:, 1, :KP1 * CH].reshape(-1)])
    return o_flat[:P, None] + bl2


# speedup vs baseline: 1.3236x; 1.3236x over previous
import functools
import jax
import jax.numpy as jnp
from jax import lax
from jax.experimental import pallas as pl
from jax.experimental.pallas import tpu as pltpu
from jax.experimental.pallas import tpu_sc as plsc

N = 10000
C = 128
E = 320000
P = 320000

NC = 2
NS = 16
NW = NC * NS
CH = 128
NCHUNK = (E + NW * CH - 1) // (NW * CH)
E_PAD = NW * CH * NCHUNK
KA0, KA1 = 115, 43
PCH = 64
KPP = 157
N_PAD = 10240
ROWS_PER_TILE = N_PAD // NS
B = 1024
GRID = N_PAD // B

def _wid():
    return lax.axis_index("s") * NC + lax.axis_index("c")


def _sc_deg_body(didx_hbm, out_hbm, acc, didx_v, ones_v):
    cid = lax.axis_index("c")
    sid = lax.axis_index("s")
    wid = _wid()

    def zfill(r, _):
        ones_v[r, :] = jnp.zeros((16,), jnp.float32)
        return 0
    lax.fori_loop(0, CH, zfill, 0)
    for t in range(ROWS_PER_TILE // CH):
        pltpu.sync_copy(ones_v, acc.at[pl.ds((sid * (ROWS_PER_TILE // CH) + t) * CH, CH)])

    def fill(r, _):
        ones_v[r, :] = jnp.zeros((16,), jnp.float32) + 1.0
        return 0
    lax.fori_loop(0, CH, fill, 0)
    plsc.subcore_barrier()

    def body(j, _):
        pltpu.sync_copy(didx_hbm.at[wid, j], didx_v)
        pltpu.sync_copy(ones_v, acc.at[didx_v], add=True)
        return 0
    lax.fori_loop(0, NCHUNK, body, 0)
    plsc.subcore_barrier()
    pltpu.sync_copy(acc.at[pl.ds(sid * ROWS_PER_TILE, ROWS_PER_TILE)],
                    out_hbm.at[cid, pl.ds(sid * ROWS_PER_TILE, ROWS_PER_TILE)])


def _sc_agg_body(u_hbm, sidx_hbm, didx_hbm, out_hbm, acc, sidx0, didx0, sidx1,
                 didx1, rows0, rows1, sem0, sem1):
    cid = lax.axis_index("c")
    sid = lax.axis_index("s")
    wid = _wid()
    nch = jnp.where(cid == 0, KA0, KA1)

    def zfill(r, _):
        for k in range(C // 16):
            rows0[r, pl.ds(k * 16, 16)] = jnp.zeros((16,), jnp.float32)
        return 0
    lax.fori_loop(0, CH, zfill, 0)
    for t in range(ROWS_PER_TILE // CH):
        pltpu.sync_copy(rows0, acc.at[pl.ds((sid * (ROWS_PER_TILE // CH) + t) * CH, CH)])
    plsc.subcore_barrier()

    def load_idx(j, si, di):
        pltpu.sync_copy(sidx_hbm.at[wid, j], si)
        pltpu.sync_copy(didx_hbm.at[wid, j], di)

    def fire(si, rows, sem):
        pltpu.async_copy(u_hbm.at[si], rows, sem)

    def wait(si, rows, sem):
        pltpu.make_async_copy(u_hbm.at[si], rows, sem).wait()

    load_idx(0, sidx0, didx0)
    fire(sidx0, rows0, sem0)

    def body(t, _):
        j0 = 2 * t
        load_idx(j0 + 1, sidx1, didx1)
        fire(sidx1, rows1, sem1)
        wait(sidx0, rows0, sem0)
        pltpu.sync_copy(rows0, acc.at[didx0], add=True)
        load_idx(j0 + 2, sidx0, didx0)
        fire(sidx0, rows0, sem0)
        wait(sidx1, rows1, sem1)
        pltpu.sync_copy(rows1, acc.at[didx1], add=True)
        return 0
    lax.fori_loop(0, (nch - 1) // 2, body, 0)
    wait(sidx0, rows0, sem0)
    pltpu.sync_copy(rows0, acc.at[didx0], add=True)

    plsc.subcore_barrier()
    pltpu.sync_copy(acc.at[pl.ds(sid * ROWS_PER_TILE, ROWS_PER_TILE)],
                    out_hbm.at[cid, pl.ds(sid * ROWS_PER_TILE, ROWS_PER_TILE)])


def _sc_pair_body(g_hbm, sidx_hbm, didx_hbm, bl1_hbm, wl2_hbm, out_hbm,
                  gsp, sidx0, didx0, sidx1, didx1, s0, d0, s1, d1,
                  out_all, bl1_v, wl2_v, sem0, sem1):
    sid = lax.axis_index("s")
    wid = _wid()
    pltpu.sync_copy(g_hbm.at[pl.ds(sid * ROWS_PER_TILE, ROWS_PER_TILE)],
                    gsp.at[pl.ds(sid * ROWS_PER_TILE, ROWS_PER_TILE)])
    pltpu.sync_copy(bl1_hbm, bl1_v)
    pltpu.sync_copy(wl2_hbm, wl2_v)
    plsc.subcore_barrier()
    lanes = lax.iota(jnp.int32, 16)

    def load_idx(j, si, di):
        pltpu.sync_copy(sidx_hbm.at[wid, j], si)
        pltpu.sync_copy(didx_hbm.at[wid, j], di)

    def fire(si, di, bs, bd, sem):
        pltpu.async_copy(gsp.at[si], bs, sem)
        pltpu.async_copy(gsp.at[di], bd, sem)

    def wait(si, di, bs, bd, sem):
        pltpu.make_async_copy(gsp.at[si], bs, sem).wait()
        pltpu.make_async_copy(gsp.at[di], bd, sem).wait()

    def compute(j, bs, bd):
        def pair16(p, _):
            tot = jnp.zeros((16,), jnp.float32)
            for ii in range(16):
                i = p * 16 + ii
                acc = jnp.zeros((16,), jnp.float32)
                for k in range(C // 16):
                    d = bd[i, pl.ds(k * 16, 16)]
                    s = bs[i, pl.ds(k * 16, 16)]
                    t = jnp.maximum(d - s + bl1_v[pl.ds(k * 16, 16)], 0.0)
                    acc = acc + t * wl2_v[pl.ds(k * 16, 16)]
                tot = jnp.where(lanes == ii, jnp.sum(acc), tot)
            out_all[pl.ds(j * PCH + p * 16, 16)] = tot
            return 0
        lax.fori_loop(0, PCH // 16, pair16, 0)

    load_idx(0, sidx0, didx0)
    fire(sidx0, didx0, s0, d0, sem0)

    def body(t, _):
        j0 = 2 * t
        load_idx(j0 + 1, sidx1, didx1)
        fire(sidx1, didx1, s1, d1, sem1)
        wait(sidx0, didx0, s0, d0, sem0)
        compute(j0, s0, d0)
        load_idx(j0 + 2, sidx0, didx0)
        fire(sidx0, didx0, s0, d0, sem0)
        wait(sidx1, didx1, s1, d1, sem1)
        compute(j0 + 1, s1, d1)
        return 0
    lax.fori_loop(0, (KPP - 1) // 2, body, 0)
    wait(sidx0, didx0, s0, d0, sem0)
    compute(KPP - 1, s0, d0)
    pltpu.sync_copy(out_all, out_hbm.at[wid])


@functools.lru_cache(maxsize=1)
def _get_sc_kernels():
    mesh = plsc.VectorSubcoreMesh(core_axis_name="c", subcore_axis_name="s",
                                  num_cores=NC, num_subcores=NS)
    cp = pltpu.CompilerParams(needs_layout_passes=False)
    deg = pl.kernel(
        _sc_deg_body,
        out_type=jax.ShapeDtypeStruct((NC, N_PAD, 16), jnp.float32),
        mesh=mesh, compiler_params=cp,
        scratch_types=[
            pltpu.VMEM_SHARED((N_PAD, 16), jnp.float32),
            pltpu.VMEM((CH,), jnp.int32),
            pltpu.VMEM((CH, 16), jnp.float32),
        ])
    agg = pl.kernel(
        _sc_agg_body,
        out_type=jax.ShapeDtypeStruct((NC, N_PAD, C), jnp.float32),
        mesh=mesh, compiler_params=cp,
        scratch_types=[
            pltpu.VMEM_SHARED((N_PAD, C), jnp.float32),
            pltpu.VMEM((CH,), jnp.int32),
            pltpu.VMEM((CH,), jnp.int32),
            pltpu.VMEM((CH,), jnp.int32),
            pltpu.VMEM((CH,), jnp.int32),
            pltpu.VMEM((CH, C), jnp.float32),
            pltpu.VMEM((CH, C), jnp.float32),
            pltpu.SemaphoreType.DMA,
            pltpu.SemaphoreType.DMA,
        ])
    pair = pl.kernel(
        _sc_pair_body,
        out_type=jax.ShapeDtypeStruct((NW, KPP * PCH), jnp.float32),
        mesh=mesh, compiler_params=cp,
        scratch_types=[
            pltpu.VMEM_SHARED((N_PAD, C), jnp.float32),
            pltpu.VMEM((PCH,), jnp.int32),
            pltpu.VMEM((PCH,), jnp.int32),
            pltpu.VMEM((PCH,), jnp.int32),
            pltpu.VMEM((PCH,), jnp.int32),
            pltpu.VMEM((PCH, C), jnp.float32),
            pltpu.VMEM((PCH, C), jnp.float32),
            pltpu.VMEM((PCH, C), jnp.float32),
            pltpu.VMEM((PCH, C), jnp.float32),
            pltpu.VMEM((KPP * PCH,), jnp.float32),
            pltpu.VMEM((C,), jnp.float32),
            pltpu.VMEM((C,), jnp.float32),
            pltpu.SemaphoreType.DMA,
            pltpu.SemaphoreType.DMA,
        ])
    return deg, agg, pair


def _tc_embed_body(x_ref, degp_ref, We1_ref, be1_ref, We2_ref, be2_ref,
                   Wr_ref, br_ref, Wc1_ref, x0_ref, u1_ref):
    xb = x_ref[...]
    h = jnp.dot(xb.reshape(B * 5, C), We1_ref[...],
                preferred_element_type=jnp.float32) + be1_ref[...]
    h = jnp.maximum(h, 0.0)
    h = jnp.dot(h, We2_ref[...], preferred_element_type=jnp.float32) + be2_ref[...]
    h = h.reshape(B, 5, C).sum(axis=1)
    h = jnp.dot(h, Wr_ref[...], preferred_element_type=jnp.float32) + br_ref[...]
    x0 = jnp.maximum(h, 0.0)
    deg = degp_ref[0, :, 0:1] + degp_ref[1, :, 0:1] + 1.0
    dinv = lax.rsqrt(deg)
    x0_ref[...] = x0
    u1_ref[...] = jnp.dot(x0, Wc1_ref[...], preferred_element_type=jnp.float32) * dinv


def _tc_mid_body(A_ref, u_ref, degp_ref, bc1_ref, Wc2_ref, u2_ref):
    deg = degp_ref[0, :, 0:1] + degp_ref[1, :, 0:1] + 1.0
    dinv = lax.rsqrt(deg)
    y = jnp.maximum(dinv * (A_ref[0] + A_ref[1] + u_ref[...]) + bc1_ref[...], 0.0)
    u2_ref[...] = jnp.dot(y, Wc2_ref[...], preferred_element_type=jnp.float32) * dinv


def _tc_fin_body(A_ref, u_ref, degp_ref, x0_ref, bc2_ref, Wl1_ref, g_ref):
    deg = degp_ref[0, :, 0:1] + degp_ref[1, :, 0:1] + 1.0
    dinv = lax.rsqrt(deg)
    y2 = dinv * (A_ref[0] + A_ref[1] + u_ref[...]) + bc2_ref[...]
    x0 = x0_ref[...]
    emb = jnp.maximum(x0 + y2, 0.0) + x0
    g_ref[...] = jnp.dot(emb, Wl1_ref[...], preferred_element_type=jnp.float32)


def _row_spec(shape3=False):
    if shape3:
        return pl.BlockSpec((B, 5, C), lambda i: (i, 0, 0))
    return pl.BlockSpec((B, C), lambda i: (i, 0))


_full = lambda s: pl.BlockSpec(s, lambda i: tuple(0 for _ in s))
_degp_spec = pl.BlockSpec((NC, B, 16), lambda i: (0, i, 0))
_A_spec = pl.BlockSpec((NC, B, C), lambda i: (0, i, 0))


def _split(idx, k0, k1, fill, ch=CH):
    n0 = NS * k0 * ch
    n1 = NS * k1 * ch
    arr = jnp.concatenate([idx, jnp.full((n0 + n1 - idx.shape[0],), fill, jnp.int32)])
    p0 = arr[:n0].reshape(NS, k0, ch)
    p1 = arr[n0:].reshape(NS, k1, ch)
    p1 = jnp.pad(p1, ((0, 0), (0, k0 - k1), (0, 0)), constant_values=fill)
    return jnp.stack([p0, p1], axis=1).reshape(NW, k0, ch)


def kernel(x, edge_index, src_idx, dst_idx, We1, be1, We2, be2, Wr, br,
           Wc1, bc1, Wc2, bc2, Wl1, bl1, Wl2, bl2):
    f32 = jnp.float32
    src = edge_index[0]
    dst = edge_index[1]
    epad = E_PAD - E
    dstp_deg = jnp.concatenate([dst, jnp.full((epad,), N, jnp.int32)]).reshape(NW, NCHUNK, CH)
    srcp = _split(src, KA0, KA1, N)
    dstp = _split(dst, KA0, KA1, N)
    sip = _split(src_idx, KPP, KPP, 0, ch=PCH)
    dip = _split(dst_idx, KPP, KPP, 0, ch=PCH)
    xp = jnp.pad(x, ((0, N_PAD - N), (0, 0), (0, 0)))

    sc_deg, sc_agg, sc_pair = _get_sc_kernels()
    degp = sc_deg(dstp_deg)

    x0, u1 = pl.pallas_call(
        _tc_embed_body,
        grid=(GRID,),
        in_specs=[_row_spec(True), _degp_spec, _full((C, C)), _full((1, C)),
                  _full((C, C)), _full((1, C)), _full((C, C)), _full((1, C)),
                  _full((C, C))],
        out_specs=[_row_spec(), _row_spec()],
        out_shape=[jax.ShapeDtypeStruct((N_PAD, C), f32),
                   jax.ShapeDtypeStruct((N_PAD, C), f32)],
    )(xp, degp, We1, be1.reshape(1, C), We2, be2.reshape(1, C),
      Wr, br.reshape(1, C), Wc1)

    A1 = sc_agg(u1, srcp, dstp)

    u2 = pl.pallas_call(
        _tc_mid_body,
        grid=(GRID,),
        in_specs=[_A_spec, _row_spec(), _degp_spec, _full((1, C)), _full((C, C))],
        out_specs=_row_spec(),
        out_shape=jax.ShapeDtypeStruct((N_PAD, C), f32),
    )(A1, u1, degp, bc1.reshape(1, C), Wc2)

    A2 = sc_agg(u2, srcp, dstp)

    g = pl.pallas_call(
        _tc_fin_body,
        grid=(GRID,),
        in_specs=[_A_spec, _row_spec(), _degp_spec, _row_spec(), _full((1, C)),
                  _full((C, C))],
        out_specs=_row_spec(),
        out_shape=jax.ShapeDtypeStruct((N_PAD, C), f32),
    )(A2, u2, degp, x0, bc2.reshape(1, C), Wl1)

    o = sc_pair(g, sip, dip, bl1, Wl2.reshape(C))
    r = o.reshape(NS, NC, KPP * PCH)
    o_flat = jnp.concatenate([r[:, 0, :].reshape(-1),
                              r[:, 1, :].reshape(-1)])
    return o_flat[:P, None] + bl2


# speedup vs baseline: 1.4037x; 1.0605x over previous
import functools
import jax
import jax.numpy as jnp
from jax import lax
from jax.experimental import pallas as pl
from jax.experimental.pallas import tpu as pltpu
from jax.experimental.pallas import tpu_sc as plsc

N = 10000
C = 128
E = 320000
P = 320000

NC = 2
NS = 16
NW = NC * NS
CH = 128
NCHUNK = (E + NW * CH - 1) // (NW * CH)
E_PAD = NW * CH * NCHUNK
KA0, KA1 = 119, 39
PCH = 64
KPP = 157
N_PAD = 10240
ROWS_PER_TILE = N_PAD // NS
B = 1024
GRID = N_PAD // B

def _wid():
    return lax.axis_index("c") * NS + lax.axis_index("s")


def _sc_deg_body(didx_hbm, out_hbm, acc, didx_v, ones_v):
    cid = lax.axis_index("c")
    sid = lax.axis_index("s")
    wid = _wid()

    def zfill(r, _):
        ones_v[r, :] = jnp.zeros((16,), jnp.float32)
        return 0
    lax.fori_loop(0, CH, zfill, 0)
    for t in range(ROWS_PER_TILE // CH):
        pltpu.sync_copy(ones_v, acc.at[pl.ds((sid * (ROWS_PER_TILE // CH) + t) * CH, CH)])

    def fill(r, _):
        ones_v[r, :] = jnp.zeros((16,), jnp.float32) + 1.0
        return 0
    lax.fori_loop(0, CH, fill, 0)
    plsc.subcore_barrier()

    def body(j, _):
        pltpu.sync_copy(didx_hbm.at[wid, j], didx_v)
        pltpu.sync_copy(ones_v, acc.at[didx_v], add=True)
        return 0
    lax.fori_loop(0, NCHUNK, body, 0)
    plsc.subcore_barrier()
    pltpu.sync_copy(acc.at[pl.ds(sid * ROWS_PER_TILE, ROWS_PER_TILE)],
                    out_hbm.at[cid, pl.ds(sid * ROWS_PER_TILE, ROWS_PER_TILE)])


def _sc_agg_body(u_hbm, sidx_hbm, didx_hbm, out_hbm, acc, sidx0, didx0, sidx1,
                 didx1, rows0, rows1, sem0, sem1):
    cid = lax.axis_index("c")
    sid = lax.axis_index("s")
    wid = _wid()
    nch = jnp.where(cid == 0, KA0, KA1)

    def zfill(r, _):
        for k in range(C // 16):
            rows0[r, pl.ds(k * 16, 16)] = jnp.zeros((16,), jnp.float32)
        return 0
    lax.fori_loop(0, CH, zfill, 0)
    for t in range(ROWS_PER_TILE // CH):
        pltpu.sync_copy(rows0, acc.at[pl.ds((sid * (ROWS_PER_TILE // CH) + t) * CH, CH)])
    plsc.subcore_barrier()

    def load_idx(j, si, di):
        pltpu.sync_copy(sidx_hbm.at[wid, j], si)
        pltpu.sync_copy(didx_hbm.at[wid, j], di)

    def fire(si, rows, sem):
        pltpu.async_copy(u_hbm.at[si], rows, sem)

    def wait(si, rows, sem):
        pltpu.make_async_copy(u_hbm.at[si], rows, sem).wait()

    load_idx(0, sidx0, didx0)
    fire(sidx0, rows0, sem0)

    def body(t, _):
        j0 = 2 * t
        load_idx(j0 + 1, sidx1, didx1)
        fire(sidx1, rows1, sem1)
        wait(sidx0, rows0, sem0)
        pltpu.sync_copy(rows0, acc.at[didx0], add=True)
        load_idx(j0 + 2, sidx0, didx0)
        fire(sidx0, rows0, sem0)
        wait(sidx1, rows1, sem1)
        pltpu.sync_copy(rows1, acc.at[didx1], add=True)
        return 0
    lax.fori_loop(0, (nch - 1) // 2, body, 0)
    wait(sidx0, rows0, sem0)
    pltpu.sync_copy(rows0, acc.at[didx0], add=True)

    plsc.subcore_barrier()
    pltpu.sync_copy(acc.at[pl.ds(sid * ROWS_PER_TILE, ROWS_PER_TILE)],
                    out_hbm.at[cid, pl.ds(sid * ROWS_PER_TILE, ROWS_PER_TILE)])


def _sc_pair_body(g_hbm, sidx_hbm, didx_hbm, bl1_hbm, wl2_hbm, out_hbm,
                  gsp, sidx0, didx0, sidx1, didx1, s0, d0, s1, d1,
                  out_all, bl1_v, wl2_v, sem0, sem1):
    sid = lax.axis_index("s")
    wid = _wid()
    pltpu.sync_copy(g_hbm.at[pl.ds(sid * ROWS_PER_TILE, ROWS_PER_TILE)],
                    gsp.at[pl.ds(sid * ROWS_PER_TILE, ROWS_PER_TILE)])
    pltpu.sync_copy(bl1_hbm, bl1_v)
    pltpu.sync_copy(wl2_hbm, wl2_v)
    plsc.subcore_barrier()
    lanes = lax.iota(jnp.int32, 16)

    def load_idx(j, si, di):
        pltpu.sync_copy(sidx_hbm.at[wid, j], si)
        pltpu.sync_copy(didx_hbm.at[wid, j], di)

    def fire(si, di, bs, bd, sem):
        pltpu.async_copy(gsp.at[si], bs, sem)
        pltpu.async_copy(gsp.at[di], bd, sem)

    def wait(si, di, bs, bd, sem):
        pltpu.make_async_copy(gsp.at[si], bs, sem).wait()
        pltpu.make_async_copy(gsp.at[di], bd, sem).wait()

    def compute(j, bs, bd):
        def pair16(p, _):
            tot = jnp.zeros((16,), jnp.float32)
            for ii in range(16):
                i = p * 16 + ii
                acc = jnp.zeros((16,), jnp.float32)
                for k in range(C // 16):
                    d = bd[i, pl.ds(k * 16, 16)]
                    s = bs[i, pl.ds(k * 16, 16)]
                    t = jnp.maximum(d - s + bl1_v[pl.ds(k * 16, 16)], 0.0)
                    acc = acc + t * wl2_v[pl.ds(k * 16, 16)]
                tot = jnp.where(lanes == ii, jnp.sum(acc), tot)
            out_all[pl.ds(j * PCH + p * 16, 16)] = tot
            return 0
        lax.fori_loop(0, PCH // 16, pair16, 0)

    load_idx(0, sidx0, didx0)
    fire(sidx0, didx0, s0, d0, sem0)

    def body(t, _):
        j0 = 2 * t
        load_idx(j0 + 1, sidx1, didx1)
        fire(sidx1, didx1, s1, d1, sem1)
        wait(sidx0, didx0, s0, d0, sem0)
        compute(j0, s0, d0)
        load_idx(j0 + 2, sidx0, didx0)
        fire(sidx0, didx0, s0, d0, sem0)
        wait(sidx1, didx1, s1, d1, sem1)
        compute(j0 + 1, s1, d1)
        return 0
    lax.fori_loop(0, (KPP - 1) // 2, body, 0)
    wait(sidx0, didx0, s0, d0, sem0)
    compute(KPP - 1, s0, d0)
    pltpu.sync_copy(out_all, out_hbm.at[wid])


@functools.lru_cache(maxsize=1)
def _get_sc_kernels():
    mesh = plsc.VectorSubcoreMesh(core_axis_name="c", subcore_axis_name="s",
                                  num_cores=NC, num_subcores=NS)
    cp = pltpu.CompilerParams(needs_layout_passes=False)
    deg = pl.kernel(
        _sc_deg_body,
        out_type=jax.ShapeDtypeStruct((NC, N_PAD, 16), jnp.float32),
        mesh=mesh, compiler_params=cp,
        scratch_types=[
            pltpu.VMEM_SHARED((N_PAD, 16), jnp.float32),
            pltpu.VMEM((CH,), jnp.int32),
            pltpu.VMEM((CH, 16), jnp.float32),
        ])
    agg = pl.kernel(
        _sc_agg_body,
        out_type=jax.ShapeDtypeStruct((NC, N_PAD, C), jnp.float32),
        mesh=mesh, compiler_params=cp,
        scratch_types=[
            pltpu.VMEM_SHARED((N_PAD, C), jnp.float32),
            pltpu.VMEM((CH,), jnp.int32),
            pltpu.VMEM((CH,), jnp.int32),
            pltpu.VMEM((CH,), jnp.int32),
            pltpu.VMEM((CH,), jnp.int32),
            pltpu.VMEM((CH, C), jnp.float32),
            pltpu.VMEM((CH, C), jnp.float32),
            pltpu.SemaphoreType.DMA,
            pltpu.SemaphoreType.DMA,
        ])
    pair = pl.kernel(
        _sc_pair_body,
        out_type=jax.ShapeDtypeStruct((NW, KPP * PCH), jnp.float32),
        mesh=mesh, compiler_params=cp,
        scratch_types=[
            pltpu.VMEM_SHARED((N_PAD, C), jnp.float32),
            pltpu.VMEM((PCH,), jnp.int32),
            pltpu.VMEM((PCH,), jnp.int32),
            pltpu.VMEM((PCH,), jnp.int32),
            pltpu.VMEM((PCH,), jnp.int32),
            pltpu.VMEM((PCH, C), jnp.float32),
            pltpu.VMEM((PCH, C), jnp.float32),
            pltpu.VMEM((PCH, C), jnp.float32),
            pltpu.VMEM((PCH, C), jnp.float32),
            pltpu.VMEM((KPP * PCH,), jnp.float32),
            pltpu.VMEM((C,), jnp.float32),
            pltpu.VMEM((C,), jnp.float32),
            pltpu.SemaphoreType.DMA,
            pltpu.SemaphoreType.DMA,
        ])
    return deg, agg, pair


def _tc_embed_body(x_ref, We1_ref, be1_ref, We2_ref, be2_ref,
                   Wr_ref, br_ref, Wc1_ref, x0_ref, xw1_ref):
    xb = x_ref[...]
    h = jnp.dot(xb.reshape(B * 5, C), We1_ref[...],
                preferred_element_type=jnp.float32) + be1_ref[...]
    h = jnp.maximum(h, 0.0)
    hs = h.reshape(B, 5, C).sum(axis=1)
    h2 = jnp.dot(hs, We2_ref[...], preferred_element_type=jnp.float32) + 5.0 * be2_ref[...]
    h2 = jnp.dot(h2, Wr_ref[...], preferred_element_type=jnp.float32) + br_ref[...]
    x0 = jnp.maximum(h2, 0.0)
    x0_ref[...] = x0
    xw1_ref[...] = jnp.dot(x0, Wc1_ref[...], preferred_element_type=jnp.float32)


def _tc_u1_body(xw1_ref, degp_ref, u1_ref):
    deg = degp_ref[0, :, 0:1] + degp_ref[1, :, 0:1] + 1.0
    dinv = lax.rsqrt(deg)
    u1_ref[...] = xw1_ref[...] * dinv


def _tc_mid_body(A_ref, u_ref, degp_ref, bc1_ref, Wc2_ref, u2_ref):
    deg = degp_ref[0, :, 0:1] + degp_ref[1, :, 0:1] + 1.0
    dinv = lax.rsqrt(deg)
    y = jnp.maximum(dinv * (A_ref[0] + A_ref[1] + u_ref[...]) + bc1_ref[...], 0.0)
    u2_ref[...] = jnp.dot(y, Wc2_ref[...], preferred_element_type=jnp.float32) * dinv


def _tc_fin_body(A_ref, u_ref, degp_ref, x0_ref, bc2_ref, Wl1_ref, g_ref):
    deg = degp_ref[0, :, 0:1] + degp_ref[1, :, 0:1] + 1.0
    dinv = lax.rsqrt(deg)
    y2 = dinv * (A_ref[0] + A_ref[1] + u_ref[...]) + bc2_ref[...]
    x0 = x0_ref[...]
    emb = jnp.maximum(x0 + y2, 0.0) + x0
    g_ref[...] = jnp.dot(emb, Wl1_ref[...], preferred_element_type=jnp.float32)


def _row_spec(shape3=False):
    if shape3:
        return pl.BlockSpec((B, 5, C), lambda i: (i, 0, 0))
    return pl.BlockSpec((B, C), lambda i: (i, 0))


_full = lambda s: pl.BlockSpec(s, lambda i: tuple(0 for _ in s))
_degp_spec = pl.BlockSpec((NC, B, 16), lambda i: (0, i, 0))
_A_spec = pl.BlockSpec((NC, B, C), lambda i: (0, i, 0))


def _split(idx, k0, k1, fill, ch=CH):
    n0 = NS * k0 * ch
    n1 = NS * k1 * ch
    arr = jnp.concatenate([idx, jnp.full((n0 + n1 - idx.shape[0],), fill, jnp.int32)])
    p0 = arr[:n0].reshape(NS, k0, ch)
    p1 = arr[n0:].reshape(NS, k1, ch)
    p1 = jnp.pad(p1, ((0, 0), (0, k0 - k1), (0, 0)), constant_values=fill)
    return jnp.concatenate([p0, p1], axis=0)


def kernel(x, edge_index, src_idx, dst_idx, We1, be1, We2, be2, Wr, br,
           Wc1, bc1, Wc2, bc2, Wl1, bl1, Wl2, bl2):
    f32 = jnp.float32
    src = edge_index[0]
    dst = edge_index[1]
    epad = E_PAD - E
    dstp_deg = jnp.concatenate([dst, jnp.full((epad,), N, jnp.int32)]).reshape(NW, NCHUNK, CH)
    srcp = _split(src, KA0, KA1, N)
    dstp = _split(dst, KA0, KA1, N)
    sip = _split(src_idx, KPP, KPP, 0, ch=PCH)
    dip = _split(dst_idx, KPP, KPP, 0, ch=PCH)
    xp = jnp.pad(x, ((0, N_PAD - N), (0, 0), (0, 0)))

    sc_deg, sc_agg, sc_pair = _get_sc_kernels()
    degp = sc_deg(dstp_deg)

    x0, xw1 = pl.pallas_call(
        _tc_embed_body,
        grid=(GRID,),
        in_specs=[_row_spec(True), _full((C, C)), _full((1, C)),
                  _full((C, C)), _full((1, C)), _full((C, C)), _full((1, C)),
                  _full((C, C))],
        out_specs=[_row_spec(), _row_spec()],
        out_shape=[jax.ShapeDtypeStruct((N_PAD, C), f32),
                   jax.ShapeDtypeStruct((N_PAD, C), f32)],
    )(xp, We1, be1.reshape(1, C), We2, be2.reshape(1, C),
      Wr, br.reshape(1, C), Wc1)

    u1 = pl.pallas_call(
        _tc_u1_body,
        grid=(GRID,),
        in_specs=[_row_spec(), _degp_spec],
        out_specs=_row_spec(),
        out_shape=jax.ShapeDtypeStruct((N_PAD, C), f32),
    )(xw1, degp)

    A1 = sc_agg(u1, srcp, dstp)

    u2 = pl.pallas_call(
        _tc_mid_body,
        grid=(GRID,),
        in_specs=[_A_spec, _row_spec(), _degp_spec, _full((1, C)), _full((C, C))],
        out_specs=_row_spec(),
        out_shape=jax.ShapeDtypeStruct((N_PAD, C), f32),
    )(A1, u1, degp, bc1.reshape(1, C), Wc2)

    A2 = sc_agg(u2, srcp, dstp)

    g = pl.pallas_call(
        _tc_fin_body,
        grid=(GRID,),
        in_specs=[_A_spec, _row_spec(), _degp_spec, _row_spec(), _full((1, C)),
                  _full((C, C))],
        out_specs=_row_spec(),
        out_shape=jax.ShapeDtypeStruct((N_PAD, C), f32),
    )(A2, u2, degp, x0, bc2.reshape(1, C), Wl1)

    o = sc_pair(g, sip, dip, bl1, Wl2.reshape(C))
    return o.reshape(-1)[:P, None] + bl2


# speedup vs baseline: 1.4458x; 1.0300x over previous
import functools
import jax
import jax.numpy as jnp
from jax import lax
from jax.experimental import pallas as pl
from jax.experimental.pallas import tpu as pltpu
from jax.experimental.pallas import tpu_sc as plsc

N = 10000
C = 128
E = 320000
P = 320000

NC = 2
NS = 16
NW = NC * NS
CH = 128
NCHUNK = (E + NW * CH - 1) // (NW * CH)
E_PAD = NW * CH * NCHUNK
KA0, KA1 = 119, 39
PCH = 64
KPP = 157
N_PAD = 10240
ROWS_PER_TILE = N_PAD // NS
B = 1024
BE = 1000
GRID = N_PAD // B

def _wid():
    return lax.axis_index("c") * NS + lax.axis_index("s")


def _sc_deg_body(didx_hbm, out_hbm, acc, didx_v, ones_v):
    cid = lax.axis_index("c")
    sid = lax.axis_index("s")
    wid = _wid()

    def zfill(r, _):
        ones_v[r, :] = jnp.zeros((16,), jnp.float32)
        return 0
    lax.fori_loop(0, CH, zfill, 0)
    for t in range(ROWS_PER_TILE // CH):
        pltpu.sync_copy(ones_v, acc.at[pl.ds((sid * (ROWS_PER_TILE // CH) + t) * CH, CH)])

    def fill(r, _):
        ones_v[r, :] = jnp.zeros((16,), jnp.float32) + 1.0
        return 0
    lax.fori_loop(0, CH, fill, 0)
    plsc.subcore_barrier()

    def body(j, _):
        pltpu.sync_copy(didx_hbm.at[wid, j], didx_v)
        pltpu.sync_copy(ones_v, acc.at[didx_v], add=True)
        return 0
    lax.fori_loop(0, NCHUNK, body, 0)
    plsc.subcore_barrier()
    pltpu.sync_copy(acc.at[pl.ds(sid * ROWS_PER_TILE, ROWS_PER_TILE)],
                    out_hbm.at[cid, pl.ds(sid * ROWS_PER_TILE, ROWS_PER_TILE)])


def _sc_agg_body(u_hbm, sidx_hbm, didx_hbm, out_hbm, acc, sidx0, didx0, sidx1,
                 didx1, rows0, rows1, sem0, sem1):
    cid = lax.axis_index("c")
    sid = lax.axis_index("s")
    wid = _wid()
    nch = jnp.where(cid == 0, KA0, KA1)

    def zfill(r, _):
        for k in range(C // 16):
            rows0[r, pl.ds(k * 16, 16)] = jnp.zeros((16,), jnp.float32)
        return 0
    lax.fori_loop(0, CH, zfill, 0)
    for t in range(ROWS_PER_TILE // CH):
        pltpu.sync_copy(rows0, acc.at[pl.ds((sid * (ROWS_PER_TILE // CH) + t) * CH, CH)])
    plsc.subcore_barrier()

    def load_idx(j, si, di):
        pltpu.sync_copy(sidx_hbm.at[wid, j], si)
        pltpu.sync_copy(didx_hbm.at[wid, j], di)

    def fire(si, rows, sem):
        pltpu.async_copy(u_hbm.at[si], rows, sem)

    def wait(si, rows, sem):
        pltpu.make_async_copy(u_hbm.at[si], rows, sem).wait()

    load_idx(0, sidx0, didx0)
    fire(sidx0, rows0, sem0)

    def body(t, _):
        j0 = 2 * t
        load_idx(j0 + 1, sidx1, didx1)
        fire(sidx1, rows1, sem1)
        wait(sidx0, rows0, sem0)
        pltpu.sync_copy(rows0, acc.at[didx0], add=True)
        load_idx(j0 + 2, sidx0, didx0)
        fire(sidx0, rows0, sem0)
        wait(sidx1, rows1, sem1)
        pltpu.sync_copy(rows1, acc.at[didx1], add=True)
        return 0
    lax.fori_loop(0, (nch - 1) // 2, body, 0)
    wait(sidx0, rows0, sem0)
    pltpu.sync_copy(rows0, acc.at[didx0], add=True)

    plsc.subcore_barrier()
    pltpu.sync_copy(acc.at[pl.ds(sid * ROWS_PER_TILE, ROWS_PER_TILE)],
                    out_hbm.at[cid, pl.ds(sid * ROWS_PER_TILE, ROWS_PER_TILE)])


def _sc_pair_body(g_hbm, pidx_hbm, bl1_hbm, wl2_hbm, out_hbm,
                  gsp, pi0, pi1, b0, b1, out_all, bl1_v, wl2_v, sem0, sem1):
    sid = lax.axis_index("s")
    wid = _wid()
    pltpu.sync_copy(g_hbm.at[pl.ds(sid * ROWS_PER_TILE, ROWS_PER_TILE)],
                    gsp.at[pl.ds(sid * ROWS_PER_TILE, ROWS_PER_TILE)])
    pltpu.sync_copy(bl1_hbm, bl1_v)
    pltpu.sync_copy(wl2_hbm, wl2_v)
    plsc.subcore_barrier()
    lanes = lax.iota(jnp.int32, 16)

    def load_idx(j, pi):
        pltpu.sync_copy(pidx_hbm.at[wid, j], pi)

    def fire(pi, b, sem):
        pltpu.async_copy(gsp.at[pi], b, sem)

    def wait(pi, b, sem):
        pltpu.make_async_copy(gsp.at[pi], b, sem).wait()

    def compute(j, b):
        def pair16(p, _):
            tot = jnp.zeros((16,), jnp.float32)
            for ii in range(16):
                i = p * 16 + ii
                acc = jnp.zeros((16,), jnp.float32)
                for k in range(C // 16):
                    d = b[PCH + i, pl.ds(k * 16, 16)]
                    s = b[i, pl.ds(k * 16, 16)]
                    t = jnp.maximum(d - s + bl1_v[pl.ds(k * 16, 16)], 0.0)
                    acc = acc + t * wl2_v[pl.ds(k * 16, 16)]
                tot = jnp.where(lanes == ii, jnp.sum(acc), tot)
            out_all[pl.ds(j * PCH + p * 16, 16)] = tot
            return 0
        lax.fori_loop(0, PCH // 16, pair16, 0)

    load_idx(0, pi0)
    fire(pi0, b0, sem0)

    def body(t, _):
        j0 = 2 * t
        load_idx(j0 + 1, pi1)
        fire(pi1, b1, sem1)
        wait(pi0, b0, sem0)
        compute(j0, b0)
        load_idx(j0 + 2, pi0)
        fire(pi0, b0, sem0)
        wait(pi1, b1, sem1)
        compute(j0 + 1, b1)
        return 0
    lax.fori_loop(0, (KPP - 1) // 2, body, 0)
    wait(pi0, b0, sem0)
    compute(KPP - 1, b0)
    pltpu.sync_copy(out_all, out_hbm.at[wid])


@functools.lru_cache(maxsize=1)
def _get_sc_kernels():
    mesh = plsc.VectorSubcoreMesh(core_axis_name="c", subcore_axis_name="s",
                                  num_cores=NC, num_subcores=NS)
    cp = pltpu.CompilerParams(needs_layout_passes=False)
    deg = pl.kernel(
        _sc_deg_body,
        out_type=jax.ShapeDtypeStruct((NC, N_PAD, 16), jnp.float32),
        mesh=mesh, compiler_params=cp,
        scratch_types=[
            pltpu.VMEM_SHARED((N_PAD, 16), jnp.float32),
            pltpu.VMEM((CH,), jnp.int32),
            pltpu.VMEM((CH, 16), jnp.float32),
        ])
    agg = pl.kernel(
        _sc_agg_body,
        out_type=jax.ShapeDtypeStruct((NC, N_PAD, C), jnp.float32),
        mesh=mesh, compiler_params=cp,
        scratch_types=[
            pltpu.VMEM_SHARED((N_PAD, C), jnp.float32),
            pltpu.VMEM((CH,), jnp.int32),
            pltpu.VMEM((CH,), jnp.int32),
            pltpu.VMEM((CH,), jnp.int32),
            pltpu.VMEM((CH,), jnp.int32),
            pltpu.VMEM((CH, C), jnp.float32),
            pltpu.VMEM((CH, C), jnp.float32),
            pltpu.SemaphoreType.DMA,
            pltpu.SemaphoreType.DMA,
        ])
    pair = pl.kernel(
        _sc_pair_body,
        out_type=jax.ShapeDtypeStruct((NW, KPP * PCH), jnp.float32),
        mesh=mesh, compiler_params=cp,
        scratch_types=[
            pltpu.VMEM_SHARED((N_PAD, C), jnp.float32),
            pltpu.VMEM((2 * PCH,), jnp.int32),
            pltpu.VMEM((2 * PCH,), jnp.int32),
            pltpu.VMEM((2 * PCH, C), jnp.float32),
            pltpu.VMEM((2 * PCH, C), jnp.float32),
            pltpu.VMEM((KPP * PCH,), jnp.float32),
            pltpu.VMEM((C,), jnp.float32),
            pltpu.VMEM((C,), jnp.float32),
            pltpu.SemaphoreType.DMA,
            pltpu.SemaphoreType.DMA,
        ])
    return deg, agg, pair


def _tc_embed_body(x_ref, We1_ref, be1_ref, We2_ref, be2_ref,
                   Wr_ref, br_ref, Wc1_ref, x0_ref, xw1_ref):
    xb = x_ref[...]
    h = jnp.dot(xb.reshape(BE * 5, C), We1_ref[...],
                preferred_element_type=jnp.float32) + be1_ref[...]
    h = jnp.maximum(h, 0.0)
    hs = h.reshape(BE, 5, C).sum(axis=1)
    h2 = jnp.dot(hs, We2_ref[...], preferred_element_type=jnp.float32) + 5.0 * be2_ref[...]
    h2 = jnp.dot(h2, Wr_ref[...], preferred_element_type=jnp.float32) + br_ref[...]
    x0 = jnp.maximum(h2, 0.0)
    x0_ref[...] = x0
    xw1_ref[...] = jnp.dot(x0, Wc1_ref[...], preferred_element_type=jnp.float32)


def _tc_u1_body(xw1_ref, degp_ref, u1_ref):
    deg = degp_ref[0, :, 0:1] + degp_ref[1, :, 0:1] + 1.0
    dinv = lax.rsqrt(deg)
    u1_ref[...] = xw1_ref[...] * dinv


def _tc_mid_body(A_ref, u_ref, degp_ref, bc1_ref, Wc2_ref, u2_ref):
    deg = degp_ref[0, :, 0:1] + degp_ref[1, :, 0:1] + 1.0
    dinv = lax.rsqrt(deg)
    y = jnp.maximum(dinv * (A_ref[0] + A_ref[1] + u_ref[...]) + bc1_ref[...], 0.0)
    u2_ref[...] = jnp.dot(y, Wc2_ref[...], preferred_element_type=jnp.float32) * dinv


def _tc_fin_body(A_ref, u_ref, degp_ref, x0_ref, bc2_ref, Wl1_ref, g_ref):
    deg = degp_ref[0, :, 0:1] + degp_ref[1, :, 0:1] + 1.0
    dinv = lax.rsqrt(deg)
    y2 = dinv * (A_ref[0] + A_ref[1] + u_ref[...]) + bc2_ref[...]
    x0 = x0_ref[...]
    emb = jnp.maximum(x0 + y2, 0.0) + x0
    g_ref[...] = jnp.dot(emb, Wl1_ref[...], preferred_element_type=jnp.float32)


def _row_spec(shape3=False):
    if shape3:
        return pl.BlockSpec((B, 5, C), lambda i: (i, 0, 0))
    return pl.BlockSpec((B, C), lambda i: (i, 0))


_full = lambda s: pl.BlockSpec(s, lambda i: tuple(0 for _ in s))
_degp_spec = pl.BlockSpec((NC, B, 16), lambda i: (0, i, 0))
_A_spec = pl.BlockSpec((NC, B, C), lambda i: (0, i, 0))


def _split(idx, k0, k1, fill, ch=CH):
    n0 = NS * k0 * ch
    n1 = NS * k1 * ch
    arr = jnp.concatenate([idx, jnp.full((n0 + n1 - idx.shape[0],), fill, jnp.int32)])
    p0 = arr[:n0].reshape(NS, k0, ch)
    p1 = arr[n0:].reshape(NS, k1, ch)
    p1 = jnp.pad(p1, ((0, 0), (0, k0 - k1), (0, 0)), constant_values=fill)
    return jnp.concatenate([p0, p1], axis=0)


def kernel(x, edge_index, src_idx, dst_idx, We1, be1, We2, be2, Wr, br,
           Wc1, bc1, Wc2, bc2, Wl1, bl1, Wl2, bl2):
    f32 = jnp.float32
    src = edge_index[0]
    dst = edge_index[1]
    epad = E_PAD - E
    dstp_deg = jnp.concatenate([dst, jnp.full((epad,), N, jnp.int32)]).reshape(NW, NCHUNK, CH)
    srcp = _split(src, KA0, KA1, N)
    dstp = _split(dst, KA0, KA1, N)
    sip = _split(src_idx, KPP, KPP, 0, ch=PCH)
    dip = _split(dst_idx, KPP, KPP, 0, ch=PCH)
    pidx = jnp.concatenate([sip[:, :, None, :], dip[:, :, None, :]],
                           axis=2).reshape(NW, KPP, 2 * PCH)

    sc_deg, sc_agg, sc_pair = _get_sc_kernels()
    degp = sc_deg(dstp_deg)

    x0r, xw1r = pl.pallas_call(
        _tc_embed_body,
        grid=(N // BE,),
        in_specs=[pl.BlockSpec((BE, 5, C), lambda i: (i, 0, 0)),
                  _full((C, C)), _full((1, C)),
                  _full((C, C)), _full((1, C)), _full((C, C)), _full((1, C)),
                  _full((C, C))],
        out_specs=[pl.BlockSpec((BE, C), lambda i: (i, 0)),
                   pl.BlockSpec((BE, C), lambda i: (i, 0))],
        out_shape=[jax.ShapeDtypeStruct((N, C), f32),
                   jax.ShapeDtypeStruct((N, C), f32)],
    )(x, We1, be1.reshape(1, C), We2, be2.reshape(1, C),
      Wr, br.reshape(1, C), Wc1)
    x0 = jnp.pad(x0r, ((0, N_PAD - N), (0, 0)))
    xw1 = jnp.pad(xw1r, ((0, N_PAD - N), (0, 0)))

    u1 = pl.pallas_call(
        _tc_u1_body,
        grid=(GRID,),
        in_specs=[_row_spec(), _degp_spec],
        out_specs=_row_spec(),
        out_shape=jax.ShapeDtypeStruct((N_PAD, C), f32),
    )(xw1, degp)

    A1 = sc_agg(u1, srcp, dstp)

    u2 = pl.pallas_call(
        _tc_mid_body,
        grid=(GRID,),
        in_specs=[_A_spec, _row_spec(), _degp_spec, _full((1, C)), _full((C, C))],
        out_specs=_row_spec(),
        out_shape=jax.ShapeDtypeStruct((N_PAD, C), f32),
    )(A1, u1, degp, bc1.reshape(1, C), Wc2)

    A2 = sc_agg(u2, srcp, dstp)

    g = pl.pallas_call(
        _tc_fin_body,
        grid=(GRID,),
        in_specs=[_A_spec, _row_spec(), _degp_spec, _row_spec(), _full((1, C)),
                  _full((C, C))],
        out_specs=_row_spec(),
        out_shape=jax.ShapeDtypeStruct((N_PAD, C), f32),
    )(A2, u2, degp, x0, bc2.reshape(1, C), Wl1)

    o = sc_pair(g, pidx, bl1, Wl2.reshape(C))
    return o.reshape(-1)[:P, None] + bl2
